# baseline (speedup 1.0000x reference)
import functools

import jax
import jax.numpy as jnp
from jax import lax
from jax.experimental import pallas as pl
from jax.experimental.pallas import tpu as pltpu
from jax.experimental.pallas import tpu_sc as plsc

VOCAB = 1000000
EMBED = 64
BATCH = 16384
NEG = 10
SLOTS = 16
ROW = 128

NUM_CORES = 2
NUM_SUBCORES = 16
LANES = 16
NW = NUM_CORES * NUM_SUBCORES
PER_W = BATCH // NW
CHUNK = 128
NCHUNKS = PER_W // CHUNK


_mesh = plsc.VectorSubcoreMesh(core_axis_name="c", subcore_axis_name="s",
                               num_cores=NUM_CORES)


@functools.partial(
    pl.kernel,
    out_type=jax.ShapeDtypeStruct((BATCH // 2, ROW), jnp.float32),
    mesh=_mesh,
    compiler_params=pltpu.CompilerParams(needs_layout_passes=False),
    scratch_types=[
        pltpu.VMEM((PER_W,), jnp.int32),
        pltpu.VMEM((PER_W,), jnp.int32),
        pltpu.VMEM((PER_W,), jnp.int32),
        pltpu.VMEM((CHUNK, ROW), jnp.float32),
        pltpu.VMEM((CHUNK // 2, ROW), jnp.float32),
        pltpu.SemaphoreType.DMA,
    ],
)
def _ei_sc(iw_hbm, win_hbm, out_hbm, iwh, iwo, raw, rows, pk, sem):
    wid = lax.axis_index("s") * NUM_CORES + lax.axis_index("c")
    wbase = wid * PER_W
    pltpu.sync_copy(iw_hbm.at[pl.ds(wbase, PER_W)], raw)
    for t in range(PER_W // LANES):
        s = pl.ds(t * LANES, LANES)
        v = raw[s]
        iwh[s] = v >> 1
        iwo[s] = (v & 1) * EMBED

    def chunk_body(ci, _):
        cbase = ci * CHUNK
        pltpu.async_copy(win_hbm.at[iwh.at[pl.ds(cbase, CHUNK)]], rows,
                         sem).wait()

        def g_body(g, _):
            gb = g * LANES
            voff = iwo[pl.ds(cbase + gb, LANES)]
            for l in range(LANES):
                j = gb + l
                off = voff[l]
                for q in range(EMBED // LANES):
                    pk[j // 2, pl.ds((j % 2) * EMBED + q * LANES, LANES)] = (
                        rows[j, pl.ds(off + q * LANES, LANES)])
            return 0

        lax.fori_loop(0, CHUNK // LANES, g_body, 0)
        orow = pl.multiple_of((wbase + cbase) // 2, 8)
        pltpu.sync_copy(pk, out_hbm.at[pl.ds(orow, CHUNK // 2)])
        return 0

    lax.fori_loop(0, NCHUNKS, chunk_body, 0)


@functools.partial(
    pl.kernel,
    out_type=jax.ShapeDtypeStruct((BATCH, SLOTS), jnp.float32),
    mesh=_mesh,
    compiler_params=pltpu.CompilerParams(needs_layout_passes=False,
                                         use_tc_tiling_on_sc=False),
    scratch_types=[
        pltpu.VMEM((CHUNK,), jnp.int32),
        pltpu.VMEM((NEG, CHUNK), jnp.int32),
        pltpu.VMEM((CHUNK // 2, ROW), jnp.float32),
        pltpu.VMEM((CHUNK, EMBED), jnp.float32),
        pltpu.VMEM((NEG, CHUNK, EMBED), jnp.float32),
        pltpu.VMEM((CHUNK, SLOTS), jnp.float32),
        pltpu.SemaphoreType.DMA,
    ],
)
def _scores_sc(cw_hbm, negt_hbm, eip_hbm, wctx_hbm, out_hbm,
               idx_ctx, idx_neg, eip, ec, en, sc_v, sem):
    wid = lax.axis_index("s") * NUM_CORES + lax.axis_index("c")
    lane = lax.iota(jnp.int32, LANES)

    def chunk_body(ci, _):
        base = wid * PER_W + ci * CHUNK
        pltpu.sync_copy(cw_hbm.at[pl.ds(base, CHUNK)], idx_ctx)
        for k in range(NEG):
            pltpu.sync_copy(negt_hbm.at[pl.ds(k * BATCH + base, CHUNK)],
                            idx_neg.at[k])
        pltpu.sync_copy(eip_hbm.at[pl.ds(base // 2, CHUNK // 2)], eip)
        copies = [pltpu.async_copy(wctx_hbm.at[idx_ctx], ec, sem)]
        for k in range(NEG):
            copies.append(pltpu.async_copy(wctx_hbm.at[idx_neg.at[k]],
                                           en.at[k], sem))
        for c in copies:
            c.wait()

        nq = EMBED // LANES

        def j_body(j, _):
            eir = [eip[j // 2, pl.ds((j % 2) * EMBED + q * LANES, LANES)]
                   for q in range(nq)]
            ecr = [ec[j, pl.ds(q * LANES, LANES)] for q in range(nq)]
            p = eir[0] * ecr[0]
            for q in range(1, nq):
                p = p + eir[q] * ecr[q]
            vals = jnp.where(lane == 0, jnp.sum(p), 0.0)
            for k in range(NEG):
                enr = [en[k, j, pl.ds(q * LANES, LANES)] for q in range(nq)]
                p = eir[0] * enr[0]
                for q in range(1, nq):
                    p = p + eir[q] * enr[q]
                vals = jnp.where(lane == k + 1, -jnp.sum(p), vals)
            sc_v[j, :] = vals
            return 0

        lax.fori_loop(0, CHUNK, j_body, 0)
        pltpu.sync_copy(sc_v, out_hbm.at[pl.ds(base, CHUNK)])
        return 0

    lax.fori_loop(0, NCHUNKS, chunk_body, 0)


def _loss_tc(scores_ref, out_ref):
    x = scores_ref[...]
    col = lax.broadcasted_iota(jnp.int32, x.shape, 1)
    ls = jnp.minimum(x, 0.0) - jnp.log1p(jnp.exp(-jnp.abs(x)))
    m = jnp.where(col < NEG + 1, ls, 0.0)
    out_ref[0, 0] = -jnp.sum(m) / BATCH


def kernel(input_word, context_word, W_in, W_ctx):
    neg = jax.random.randint(jax.random.key(1234), (BATCH, NEG), 0, VOCAB)
    negt = neg.astype(jnp.int32).T.reshape(-1)
    iw = input_word.astype(jnp.int32)
    cw = context_word.astype(jnp.int32)
    one = (jnp.minimum(iw[0], 0) + 1).astype(jnp.float32)
    win2 = W_in.reshape(VOCAB // 2, ROW) * one
    ei_packed = _ei_sc(iw, win2)
    scores = _scores_sc(cw, negt, ei_packed, W_ctx)
    loss = pl.pallas_call(
        _loss_tc,
        out_shape=jax.ShapeDtypeStruct((1, 1), jnp.float32),
        out_specs=pl.BlockSpec(memory_space=pltpu.SMEM),
    )(scores)
    return loss[0, 0]


# speedup vs baseline: 1.0017x; 1.0017x over previous
import functools

import jax
import jax.numpy as jnp
from jax import lax
from jax.experimental import pallas as pl
from jax.experimental.pallas import tpu as pltpu
from jax.experimental.pallas import tpu_sc as plsc

VOCAB = 1000000
EMBED = 64
BATCH = 16384
NEG = 10
SLOTS = 16
ROW = 128

NUM_CORES = 2
NUM_SUBCORES = 16
LANES = 16
NW = NUM_CORES * NUM_SUBCORES
PER_W = BATCH // NW
CHUNK = 128
NCHUNKS = PER_W // CHUNK


_mesh = plsc.VectorSubcoreMesh(core_axis_name="c", subcore_axis_name="s",
                               num_cores=NUM_CORES)


@functools.partial(
    pl.kernel,
    out_type=jax.ShapeDtypeStruct((BATCH // 2, ROW), jnp.float32),
    mesh=_mesh,
    compiler_params=pltpu.CompilerParams(needs_layout_passes=False),
    scratch_types=[
        pltpu.VMEM((PER_W,), jnp.int32),
        pltpu.VMEM((PER_W,), jnp.int32),
        pltpu.VMEM((PER_W,), jnp.int32),
        pltpu.VMEM((CHUNK, ROW), jnp.float32),
        pltpu.VMEM((CHUNK // 2, ROW), jnp.float32),
        pltpu.SemaphoreType.DMA,
    ],
)
def _ei_sc(iw_hbm, win_hbm, out_hbm, iwh, iwo, raw, rows, pk, sem):
    wid = lax.axis_index("s") * NUM_CORES + lax.axis_index("c")
    wbase = wid * PER_W
    pltpu.sync_copy(iw_hbm.at[pl.ds(wbase, PER_W)], raw)
    for t in range(PER_W // LANES):
        s = pl.ds(t * LANES, LANES)
        v = raw[s]
        iwh[s] = v >> 1
        iwo[s] = (v & 1) * EMBED

    def chunk_body(ci, _):
        cbase = ci * CHUNK
        pltpu.async_copy(win_hbm.at[iwh.at[pl.ds(cbase, CHUNK)]], rows,
                         sem).wait()

        def g_body(g, _):
            gb = g * LANES
            voff = iwo[pl.ds(cbase + gb, LANES)]
            for l in range(LANES):
                j = gb + l
                off = voff[l]
                for q in range(EMBED // LANES):
                    pk[j // 2, pl.ds((j % 2) * EMBED + q * LANES, LANES)] = (
                        rows[j, pl.ds(off + q * LANES, LANES)])
            return 0

        lax.fori_loop(0, CHUNK // LANES, g_body, 0)
        orow = pl.multiple_of((wbase + cbase) // 2, 8)
        pltpu.sync_copy(pk, out_hbm.at[pl.ds(orow, CHUNK // 2)])
        return 0

    lax.fori_loop(0, NCHUNKS, chunk_body, 0)


@functools.partial(
    pl.kernel,
    out_type=jax.ShapeDtypeStruct((BATCH, SLOTS), jnp.float32),
    mesh=_mesh,
    compiler_params=pltpu.CompilerParams(needs_layout_passes=False,
                                         use_tc_tiling_on_sc=False),
    scratch_types=[
        pltpu.VMEM((CHUNK,), jnp.int32),
        pltpu.VMEM((NEG, CHUNK), jnp.int32),
        pltpu.VMEM((CHUNK // 2, ROW), jnp.float32),
        pltpu.VMEM((CHUNK, EMBED), jnp.float32),
        pltpu.VMEM((NEG, CHUNK, EMBED), jnp.float32),
        pltpu.VMEM((CHUNK, SLOTS), jnp.float32),
        pltpu.SemaphoreType.DMA,
    ],
)
def _scores_sc(cw_hbm, negt_hbm, eip_hbm, wctx_hbm, out_hbm,
               idx_ctx, idx_neg, eip, ec, en, sc_v, sem):
    wid = lax.axis_index("s") * NUM_CORES + lax.axis_index("c")
    lane = lax.iota(jnp.int32, LANES)

    def chunk_body(ci, _):
        base = wid * PER_W + ci * CHUNK
        pltpu.sync_copy(cw_hbm.at[pl.ds(base, CHUNK)], idx_ctx)
        for k in range(NEG):
            pltpu.sync_copy(negt_hbm.at[pl.ds(k * BATCH + base, CHUNK)],
                            idx_neg.at[k])
        pltpu.sync_copy(eip_hbm.at[pl.ds(base // 2, CHUNK // 2)], eip)
        copies = [pltpu.async_copy(wctx_hbm.at[idx_ctx], ec, sem)]
        for k in range(NEG):
            copies.append(pltpu.async_copy(wctx_hbm.at[idx_neg.at[k]],
                                           en.at[k], sem))
        for c in copies:
            c.wait()

        nq = EMBED // LANES

        def j_body(j, _):
            eir = [eip[j // 2, pl.ds((j % 2) * EMBED + q * LANES, LANES)]
                   for q in range(nq)]
            ecr = [ec[j, pl.ds(q * LANES, LANES)] for q in range(nq)]
            p = eir[0] * ecr[0]
            for q in range(1, nq):
                p = p + eir[q] * ecr[q]
            vals = jnp.where(lane == 0, jnp.sum(p), 0.0)
            for k in range(NEG):
                enr = [en[k, j, pl.ds(q * LANES, LANES)] for q in range(nq)]
                p = eir[0] * enr[0]
                for q in range(1, nq):
                    p = p + eir[q] * enr[q]
                vals = jnp.where(lane == k + 1, -jnp.sum(p), vals)
            sc_v[j, :] = vals
            return 0

        lax.fori_loop(0, CHUNK, j_body, 0)
        pltpu.sync_copy(sc_v, out_hbm.at[pl.ds(base, CHUNK)])
        return 0

    lax.fori_loop(0, NCHUNKS, chunk_body, 0)


def _loss_tc(scores_ref, out_ref):
    x = scores_ref[...]
    col = lax.broadcasted_iota(jnp.int32, x.shape, 1)
    ls = jnp.minimum(x, 0.0) - jnp.log1p(jnp.exp(-jnp.abs(x)))
    m = jnp.where(col < NEG + 1, ls, 0.0)
    out_ref[0, 0] = -jnp.sum(m) / BATCH


def kernel(input_word, context_word, W_in, W_ctx):
    neg = jax.random.randint(jax.random.key(1234), (BATCH, NEG), 0, VOCAB)
    negt = neg.astype(jnp.int32).T.reshape(-1)
    iw = input_word.astype(jnp.int32)
    cw = context_word.astype(jnp.int32)
    one = (jnp.minimum(iw[0], 0) + 1).astype(jnp.float32)
    from jax.experimental.layout import Format, Layout, with_layout_constraint
    win2 = with_layout_constraint(W_in.reshape(VOCAB // 2, ROW) * one,
                                  Layout((0, 1)))
    ei_packed = _ei_sc(iw, win2)
    scores = _scores_sc(cw, negt, ei_packed, W_ctx)
    loss = pl.pallas_call(
        _loss_tc,
        out_shape=jax.ShapeDtypeStruct((1, 1), jnp.float32),
        out_specs=pl.BlockSpec(memory_space=pltpu.SMEM),
    )(scores)
    return loss[0, 0]


# speedup vs baseline: 1.1401x; 1.1382x over previous
import functools

import jax
import jax.numpy as jnp
from jax import lax
from jax.experimental import pallas as pl
from jax.experimental.pallas import tpu as pltpu
from jax.experimental.pallas import tpu_sc as plsc

VOCAB = 1000000
EMBED = 64
BATCH = 16384
NEG = 10
SLOTS = 16

NUM_CORES = 2
NUM_SUBCORES = 16
LANES = 16
NW = NUM_CORES * NUM_SUBCORES
PER_W = BATCH // NW
CHUNK = 128
NCHUNKS = PER_W // CHUNK


_mesh = plsc.VectorSubcoreMesh(core_axis_name="c", subcore_axis_name="s",
                               num_cores=NUM_CORES)


@functools.partial(
    pl.kernel,
    out_type=jax.ShapeDtypeStruct((BATCH, SLOTS), jnp.float32),
    mesh=_mesh,
    compiler_params=pltpu.CompilerParams(needs_layout_passes=False,
                                         use_tc_tiling_on_sc=False),
    scratch_types=[
        pltpu.VMEM((CHUNK,), jnp.int32),
        pltpu.VMEM((CHUNK,), jnp.int32),
        pltpu.VMEM((NEG, CHUNK), jnp.int32),
        pltpu.VMEM((CHUNK, EMBED), jnp.float32),
        pltpu.VMEM((CHUNK, EMBED), jnp.float32),
        pltpu.VMEM((NEG, CHUNK, EMBED), jnp.float32),
        pltpu.VMEM((CHUNK, SLOTS), jnp.float32),
        pltpu.SemaphoreType.DMA,
    ],
)
def _scores_sc(iw_hbm, cw_hbm, negt_hbm, win_hbm, wctx_hbm, out_hbm,
               idx_in, idx_ctx, idx_neg, ei, ec, en, sc_v, sem):
    wid = lax.axis_index("s") * NUM_CORES + lax.axis_index("c")
    lane = lax.iota(jnp.int32, LANES)

    def chunk_body(ci, _):
        base = wid * PER_W + ci * CHUNK
        pltpu.sync_copy(iw_hbm.at[pl.ds(base, CHUNK)], idx_in)
        pltpu.sync_copy(cw_hbm.at[pl.ds(base, CHUNK)], idx_ctx)
        for k in range(NEG):
            pltpu.sync_copy(negt_hbm.at[pl.ds(k * BATCH + base, CHUNK)],
                            idx_neg.at[k])
        copies = [
            pltpu.async_copy(win_hbm.at[idx_in], ei, sem),
            pltpu.async_copy(wctx_hbm.at[idx_ctx], ec, sem),
        ]
        for k in range(NEG):
            copies.append(pltpu.async_copy(wctx_hbm.at[idx_neg.at[k]],
                                           en.at[k], sem))
        for c in copies:
            c.wait()

        nq = EMBED // LANES

        def j_body(j, _):
            eir = [ei[j, pl.ds(q * LANES, LANES)] for q in range(nq)]
            ecr = [ec[j, pl.ds(q * LANES, LANES)] for q in range(nq)]
            p = eir[0] * ecr[0]
            for q in range(1, nq):
                p = p + eir[q] * ecr[q]
            vals = jnp.where(lane == 0, jnp.sum(p), 0.0)
            for k in range(NEG):
                enr = [en[k, j, pl.ds(q * LANES, LANES)] for q in range(nq)]
                p = eir[0] * enr[0]
                for q in range(1, nq):
                    p = p + eir[q] * enr[q]
                vals = jnp.where(lane == k + 1, -jnp.sum(p), vals)
            sc_v[j, :] = vals
            return 0

        lax.fori_loop(0, CHUNK, j_body, 0)
        pltpu.sync_copy(sc_v, out_hbm.at[pl.ds(base, CHUNK)])
        return 0

    lax.fori_loop(0, NCHUNKS, chunk_body, 0)


def _loss_tc(scores_ref, out_ref):
    x = scores_ref[...]
    col = lax.broadcasted_iota(jnp.int32, x.shape, 1)
    ls = jnp.minimum(x, 0.0) - jnp.log1p(jnp.exp(-jnp.abs(x)))
    m = jnp.where(col < NEG + 1, ls, 0.0)
    out_ref[0, 0] = -jnp.sum(m) / BATCH


def kernel(input_word, context_word, W_in, W_ctx):
    neg = jax.random.randint(jax.random.key(1234), (BATCH, NEG), 0, VOCAB)
    negt = neg.astype(jnp.int32).T.reshape(-1)
    iw = input_word.astype(jnp.int32)
    cw = context_word.astype(jnp.int32)
    scores = _scores_sc(iw, cw, negt, W_in, W_ctx)
    loss = pl.pallas_call(
        _loss_tc,
        out_shape=jax.ShapeDtypeStruct((1, 1), jnp.float32),
        out_specs=pl.BlockSpec(memory_space=pltpu.SMEM),
    )(scores)
    return loss[0, 0]
